# Initial kernel scaffold; baseline (speedup 1.0000x reference)
import jax
import jax.numpy as jnp
from jax.experimental import pallas as pl


def kernel(z, embedding):
    raise NotImplementedError("write your pallas kernel here")


# speedup vs baseline: 65.8228x; 65.8228x over previous
import jax
import jax.numpy as jnp
from jax import lax
from jax.experimental import pallas as pl
from jax.experimental.pallas import tpu as pltpu

_N_E = 8192
_E_DIM = 64
_ROWS = 4608
_T = 128
_GRID = _ROWS // _T


def _vq_body(z_ref, emb_ref, embT_ref, enc_ref, quant_ref, idx_ref, perp_ref,
             counts_ref):
    i = pl.program_id(0)
    zt = z_ref[...]
    zn = zt / jnp.clip(jnp.sqrt(jnp.sum(zt * zt, axis=1, keepdims=True)), 1e-12)
    z2 = jnp.sum(zn * zn, axis=1, keepdims=True)
    embT = embT_ref[...]
    embT_n = embT / jnp.clip(
        jnp.sqrt(jnp.sum(embT * embT, axis=0, keepdims=True)), 1e-12)
    e2 = jnp.sum(embT_n * embT_n, axis=0, keepdims=True)
    mm = jnp.dot(zn, embT_n, preferred_element_type=jnp.float32)
    d = (-z2 - e2) + 2.0 * mm
    m = jnp.max(d, axis=1, keepdims=True)
    iota = lax.broadcasted_iota(jnp.int32, (_T, _N_E), 1)
    idx = jnp.max(jnp.where(d == m, iota, -1), axis=1, keepdims=True)
    oh = jnp.where(iota == idx, 1.0, 0.0).astype(jnp.float32)
    enc_ref[...] = oh
    idx_ref[...] = idx
    emb = emb_ref[...]
    emb_n = emb / jnp.clip(
        jnp.sqrt(jnp.sum(emb * emb, axis=1, keepdims=True)), 1e-12)
    zq = jnp.dot(oh, emb_n, preferred_element_type=jnp.float32)
    quant_ref[...] = zn + (zq - zn)

    @pl.when(i == 0)
    def _init():
        counts_ref[...] = jnp.zeros_like(counts_ref)

    counts_ref[...] += jnp.sum(oh, axis=0, keepdims=True)

    @pl.when(i == _GRID - 1)
    def _fin():
        p = counts_ref[...] / _ROWS
        ent = jnp.sum(p * jnp.log(p + 1e-10), axis=1, keepdims=True)
        perp_ref[...] = jnp.exp(-ent)


def kernel(z, embedding):
    zt = jnp.transpose(z, (0, 2, 1)).reshape(-1, _E_DIM)
    embT = embedding.T
    enc, quant, idx, perp = pl.pallas_call(
        _vq_body,
        grid=(_GRID,),
        in_specs=[
            pl.BlockSpec((_T, _E_DIM), lambda i: (i, 0)),
            pl.BlockSpec((_N_E, _E_DIM), lambda i: (0, 0)),
            pl.BlockSpec((_E_DIM, _N_E), lambda i: (0, 0)),
        ],
        out_specs=[
            pl.BlockSpec((_T, _N_E), lambda i: (i, 0)),
            pl.BlockSpec((_T, _E_DIM), lambda i: (i, 0)),
            pl.BlockSpec((_T, 1), lambda i: (i, 0)),
            pl.BlockSpec((1, 1), lambda i: (0, 0)),
        ],
        out_shape=[
            jax.ShapeDtypeStruct((_ROWS, _N_E), jnp.float32),
            jax.ShapeDtypeStruct((_ROWS, _E_DIM), jnp.float32),
            jax.ShapeDtypeStruct((_ROWS, 1), jnp.int32),
            jax.ShapeDtypeStruct((1, 1), jnp.float32),
        ],
        scratch_shapes=[pltpu.VMEM((1, _N_E), jnp.float32)],
    )(zt, embedding, embT)
    quant = jnp.transpose(quant.reshape(z.shape[0], -1, _E_DIM), (0, 2, 1))
    zero = jnp.float32(0.0)
    return (quant, zero, zero, zero, zero, perp.reshape(()), enc,
            idx.reshape(_ROWS))


# speedup vs baseline: 89.8100x; 1.3644x over previous
import jax
import jax.numpy as jnp
from jax import lax
from jax.experimental import pallas as pl
from jax.experimental.pallas import tpu as pltpu

_N_E = 8192
_E_DIM = 64
_ROWS = 4608
_T = 256
_GRID = _ROWS // _T


def _vq_body(z_ref, emb_ref, embT_ref, enc_ref, quant_ref, idx_ref, perp_ref,
             counts_ref, embn_ref, embTn_ref, e2_ref):
    i = pl.program_id(0)

    @pl.when(i == 0)
    def _init():
        embT = embT_ref[...]
        embTn = embT / jnp.clip(
            jnp.sqrt(jnp.sum(embT * embT, axis=0, keepdims=True)), 1e-12)
        embTn_ref[...] = embTn
        e2_ref[...] = jnp.sum(embTn * embTn, axis=0, keepdims=True)
        emb = emb_ref[...]
        embn_ref[...] = emb / jnp.clip(
            jnp.sqrt(jnp.sum(emb * emb, axis=1, keepdims=True)), 1e-12)
        counts_ref[...] = jnp.zeros_like(counts_ref)

    zt = z_ref[...]
    zn = zt / jnp.clip(jnp.sqrt(jnp.sum(zt * zt, axis=1, keepdims=True)), 1e-12)
    z2 = jnp.sum(zn * zn, axis=1, keepdims=True)
    mm = jnp.dot(zn, embTn_ref[...], preferred_element_type=jnp.float32)
    d = (-z2 - e2_ref[...]) + 2.0 * mm
    m = jnp.max(d, axis=1, keepdims=True)
    iota = lax.broadcasted_iota(jnp.int32, (_T, _N_E), 1)
    idx = jnp.max(jnp.where(d == m, iota, -1), axis=1, keepdims=True)
    oh = jnp.where(iota == idx, 1.0, 0.0).astype(jnp.float32)
    enc_ref[...] = oh
    idx_ref[...] = idx
    zq = jnp.dot(oh, embn_ref[...], preferred_element_type=jnp.float32)
    quant_ref[...] = zn + (zq - zn)
    counts_ref[...] += jnp.sum(oh, axis=0, keepdims=True)

    @pl.when(i == _GRID - 1)
    def _fin():
        p = counts_ref[...] / _ROWS
        ent = jnp.sum(p * jnp.log(p + 1e-10), axis=1, keepdims=True)
        perp_ref[...] = jnp.exp(-ent)


def kernel(z, embedding):
    zt = jnp.transpose(z, (0, 2, 1)).reshape(-1, _E_DIM)
    embT = embedding.T
    enc, quant, idx, perp = pl.pallas_call(
        _vq_body,
        grid=(_GRID,),
        in_specs=[
            pl.BlockSpec((_T, _E_DIM), lambda i: (i, 0)),
            pl.BlockSpec((_N_E, _E_DIM), lambda i: (0, 0)),
            pl.BlockSpec((_E_DIM, _N_E), lambda i: (0, 0)),
        ],
        out_specs=[
            pl.BlockSpec((_T, _N_E), lambda i: (i, 0)),
            pl.BlockSpec((_T, _E_DIM), lambda i: (i, 0)),
            pl.BlockSpec((_T, 1), lambda i: (i, 0)),
            pl.BlockSpec((1, 1), lambda i: (0, 0)),
        ],
        out_shape=[
            jax.ShapeDtypeStruct((_ROWS, _N_E), jnp.float32),
            jax.ShapeDtypeStruct((_ROWS, _E_DIM), jnp.float32),
            jax.ShapeDtypeStruct((_ROWS, 1), jnp.int32),
            jax.ShapeDtypeStruct((1, 1), jnp.float32),
        ],
        scratch_shapes=[
            pltpu.VMEM((1, _N_E), jnp.float32),
            pltpu.VMEM((_N_E, _E_DIM), jnp.float32),
            pltpu.VMEM((_E_DIM, _N_E), jnp.float32),
            pltpu.VMEM((1, _N_E), jnp.float32),
        ],
    )(zt, embedding, embT)
    quant = jnp.transpose(quant.reshape(z.shape[0], -1, _E_DIM), (0, 2, 1))
    zero = jnp.float32(0.0)
    return (quant, zero, zero, zero, zero, perp.reshape(()), enc,
            idx.reshape(_ROWS))
